# baseline (speedup 1.0000x reference)
import functools

import jax
import jax.numpy as jnp
from jax import lax
from jax.experimental import pallas as pl
from jax.experimental.pallas import tpu as pltpu
from jax.experimental.pallas import tpu_sc as plsc

N_NODES = 10000
N_EDGES = 320000
D = 128

NC = 2
NS = 16
NW = NC * NS
CHUNK = 128
CPR = 160
CPT0 = 80
CPT1 = CPR - CPT0
NB = 2
G = 16
R_PAD = 10240
RPT = R_PAD // NS


def _make_sc_segment_sum(with_deg: bool):
    mesh = plsc.VectorSubcoreMesh(core_axis_name="c", subcore_axis_name="s")
    out_type = [jax.ShapeDtypeStruct((NC, R_PAD, D), jnp.float32)]
    scratch = [
        pltpu.VMEM_SHARED((R_PAD, D), jnp.float32),
        pltpu.VMEM((G, CHUNK), jnp.int32),
        pltpu.VMEM((G, CHUNK), jnp.int32),
        [pltpu.VMEM((CHUNK, D), jnp.float32)] * NB,
        [pltpu.SemaphoreType.DMA] * NB,
        [pltpu.SemaphoreType.DMA] * NB,
        pltpu.SemaphoreType.DMA,
    ]
    if with_deg:
        out_type.append(jax.ShapeDtypeStruct((NC, R_PAD, 16), jnp.float32))
        scratch += [
            pltpu.VMEM_SHARED((R_PAD, 16), jnp.float32),
            pltpu.VMEM((CHUNK, 16), jnp.float32),
        ]

    def body(m_hbm, src_hbm, dst_hbm, acc_out, *rest):
        if with_deg:
            (deg_out, acc_sh, src_v, dst_v, rows, gsems, ssems, dsem,
             deg_sh, ones_v) = rest
        else:
            (acc_sh, src_v, dst_v, rows, gsems, ssems, dsem) = rest
        cid = lax.axis_index("c")
        sid = lax.axis_index("s")
        base = sid * RPT

        @pl.loop(0, CHUNK)
        def _(i):
            @pl.loop(0, D, step=16)
            def _(j):
                rows[0][i, pl.ds(j, 16)] = jnp.zeros((16,), jnp.float32)

        @pl.loop(0, RPT, step=CHUNK)
        def _(r):
            pltpu.sync_copy(rows[0], acc_sh.at[pl.ds(base + r, CHUNK)])

        if with_deg:
            @pl.loop(0, CHUNK)
            def _(i):
                ones_v[i, pl.ds(0, 16)] = jnp.zeros((16,), jnp.float32)

            @pl.loop(0, RPT, step=CHUNK)
            def _(r):
                pltpu.sync_copy(ones_v, deg_sh.at[pl.ds(base + r, CHUNK)])

            @pl.loop(0, CHUNK)
            def _(i):
                ones_v[i, pl.ds(0, 16)] = jnp.ones((16,), jnp.float32)

        plsc.subcore_barrier()

        def edge_loop(c_lo, c_hi):
            @pl.loop(c_lo, c_hi, step=G)
            def _(g0):
                pltpu.sync_copy(src_hbm.at[sid, pl.ds(g0, G)], src_v)
                pltpu.sync_copy(dst_hbm.at[sid, pl.ds(g0, G)], dst_v)
                for b in range(NB):
                    pltpu.async_copy(m_hbm.at[src_v.at[b]], rows[b], gsems[b])

                @pl.loop(0, G - NB, step=NB)
                def _(j):
                    for b in range(NB):
                        pltpu.make_async_copy(
                            m_hbm.at[src_v.at[j + b]], rows[b],
                            gsems[b]).wait()
                        pltpu.async_copy(
                            rows[b], acc_sh.at[dst_v.at[j + b]], ssems[b],
                            add=True)
                        if with_deg:
                            pltpu.async_copy(
                                ones_v, deg_sh.at[dst_v.at[j + b]], dsem,
                                add=True)
                    for b in range(NB):
                        pltpu.make_async_copy(
                            rows[b], acc_sh.at[dst_v.at[j + b]],
                            ssems[b]).wait()
                        pltpu.async_copy(
                            m_hbm.at[src_v.at[j + NB + b]], rows[b], gsems[b])
                        if with_deg:
                            pltpu.make_async_copy(
                                ones_v, deg_sh.at[dst_v.at[j + b]],
                                dsem).wait()

                je = G - NB
                tail = []
                for b in range(NB):
                    pltpu.make_async_copy(
                        m_hbm.at[src_v.at[je + b]], rows[b], gsems[b]).wait()
                    tail.append(pltpu.async_copy(
                        rows[b], acc_sh.at[dst_v.at[je + b]], ssems[b],
                        add=True))
                    if with_deg:
                        tail.append(pltpu.async_copy(
                            ones_v, deg_sh.at[dst_v.at[je + b]], dsem,
                            add=True))
                for cp in tail:
                    cp.wait()

        with jax.named_scope("edge_loop"):
            @pl.when(cid == 0)
            def _():
                edge_loop(0, CPT0)

            @pl.when(cid == 1)
            def _():
                edge_loop(CPT0, CPR)

        plsc.subcore_barrier()

        with jax.named_scope("writeback"):
            @pl.loop(0, RPT, step=CHUNK)
            def _(r):
                pltpu.sync_copy(acc_sh.at[pl.ds(base + r, CHUNK)], rows[0])
                pltpu.sync_copy(rows[0],
                                acc_out.at[cid, pl.ds(base + r, CHUNK)])

            if with_deg:
                @pl.loop(0, RPT, step=CHUNK)
                def _(r):
                    pltpu.sync_copy(deg_sh.at[pl.ds(base + r, CHUNK)], ones_v)
                    pltpu.sync_copy(ones_v,
                                    deg_out.at[cid, pl.ds(base + r, CHUNK)])

    return pl.kernel(
        body, out_type=out_type, mesh=mesh, scratch_types=scratch,
        compiler_params=pltpu.CompilerParams(use_tc_tiling_on_sc=False))


_BR = 2000


def _tc_layer_in(x, w_self, w_neigh, b):
    n = x.shape[0]

    def body(x_ref, ws_ref, wn_ref, b_ref, s_ref, m_ref):
        xb = x_ref[...]
        s_ref[...] = (jnp.dot(xb, ws_ref[...],
                              preferred_element_type=jnp.float32) + b_ref[...])
        m_ref[...] = jnp.dot(xb, wn_ref[...],
                             preferred_element_type=jnp.float32)

    return pl.pallas_call(
        body,
        grid=(n // _BR,),
        in_specs=[
            pl.BlockSpec((_BR, D), lambda i: (i, 0)),
            pl.BlockSpec((D, D), lambda i: (0, 0)),
            pl.BlockSpec((D, D), lambda i: (0, 0)),
            pl.BlockSpec((1, D), lambda i: (0, 0)),
        ],
        out_specs=[
            pl.BlockSpec((_BR, D), lambda i: (i, 0)),
            pl.BlockSpec((_BR, D), lambda i: (i, 0)),
        ],
        out_shape=[jax.ShapeDtypeStruct((n, D), jnp.float32)] * 2,
    )(x, w_self, w_neigh, b.reshape(1, D))


def _tc_layer_mid(s1, accp, degp, w_self, w_neigh, b):
    n = s1.shape[0]

    def body(s1_ref, acc_ref, deg_ref, ws_ref, wn_ref, b_ref, s_ref, m_ref):
        agg = acc_ref[0] + acc_ref[1]
        deg = deg_ref[0][:, 0:1] + deg_ref[1][:, 0:1]
        rdeg = 1.0 / jnp.maximum(deg, 1.0)
        h = jnp.maximum(s1_ref[...] + agg * rdeg, 0.0)
        s_ref[...] = (jnp.dot(h, ws_ref[...],
                              preferred_element_type=jnp.float32) + b_ref[...])
        m_ref[...] = jnp.dot(h, wn_ref[...],
                             preferred_element_type=jnp.float32)

    return pl.pallas_call(
        body,
        grid=(n // _BR,),
        in_specs=[
            pl.BlockSpec((_BR, D), lambda i: (i, 0)),
            pl.BlockSpec((NC, _BR, D), lambda i: (0, i, 0)),
            pl.BlockSpec((NC, _BR, 16), lambda i: (0, i, 0)),
            pl.BlockSpec((D, D), lambda i: (0, 0)),
            pl.BlockSpec((D, D), lambda i: (0, 0)),
            pl.BlockSpec((1, D), lambda i: (0, 0)),
        ],
        out_specs=[
            pl.BlockSpec((_BR, D), lambda i: (i, 0)),
            pl.BlockSpec((_BR, D), lambda i: (i, 0)),
        ],
        out_shape=[jax.ShapeDtypeStruct((n, D), jnp.float32)] * 2,
    )(s1, accp, degp, w_self, w_neigh, b.reshape(1, D))


def _tc_layer_out(s2, accp, degp):
    n = s2.shape[0]

    def body(s2_ref, acc_ref, deg_ref, o_ref):
        agg = acc_ref[0] + acc_ref[1]
        deg = deg_ref[0][:, 0:1] + deg_ref[1][:, 0:1]
        rdeg = 1.0 / jnp.maximum(deg, 1.0)
        o_ref[...] = s2_ref[...] + agg * rdeg

    return pl.pallas_call(
        body,
        grid=(n // _BR,),
        in_specs=[
            pl.BlockSpec((_BR, D), lambda i: (i, 0)),
            pl.BlockSpec((NC, _BR, D), lambda i: (0, i, 0)),
            pl.BlockSpec((NC, _BR, 16), lambda i: (0, i, 0)),
        ],
        out_specs=pl.BlockSpec((_BR, D), lambda i: (i, 0)),
        out_shape=jax.ShapeDtypeStruct((n, D), jnp.float32),
    )(s2, accp, degp)


def kernel(x, edge_index, W_self1, W_neigh1, b1, W_self2, W_neigh2, b2):
    src = edge_index[0].astype(jnp.int32)
    dst = edge_index[1].astype(jnp.int32)
    pad = NS * CPR * CHUNK - src.shape[0]
    pad_iota = jnp.arange(pad, dtype=jnp.int32)
    pad_src = pad_iota % N_NODES
    pad_dst = N_NODES + pad_iota % (R_PAD - N_NODES)
    src_r = jnp.concatenate([src, pad_src]).reshape(NS, CPR, CHUNK)
    dst_r = jnp.concatenate([dst, pad_dst]).reshape(NS, CPR, CHUNK)

    s1, m1 = _tc_layer_in(x, W_self1, W_neigh1, b1)
    accp1, degp = _make_sc_segment_sum(True)(m1, src_r, dst_r)
    s2, m2 = _tc_layer_mid(s1, accp1, degp, W_self2, W_neigh2, b2)
    accp2, = _make_sc_segment_sum(False)(m2, src_r, dst_r)
    return _tc_layer_out(s2, accp2, degp)


# speedup vs baseline: 1.1413x; 1.1413x over previous
import functools

import jax
import jax.numpy as jnp
from jax import lax
from jax.experimental import pallas as pl
from jax.experimental.pallas import tpu as pltpu
from jax.experimental.pallas import tpu_sc as plsc

N_NODES = 10000
N_EDGES = 320000
D = 128

NC = 2
NS = 16
NW = NC * NS
CHUNK = 128
CPR = 160
CPT0 = 80
CPT1 = CPR - CPT0
NB = 2
G = 16
R_PAD = 10240
RPT = R_PAD // NS


def _make_sc_segment_sum(with_deg: bool):
    mesh = plsc.VectorSubcoreMesh(core_axis_name="c", subcore_axis_name="s")
    out_type = [jax.ShapeDtypeStruct((NC, R_PAD, D), jnp.float32)]
    scratch = [
        pltpu.VMEM_SHARED((R_PAD, D), jnp.float32),
        pltpu.VMEM((G, CHUNK), jnp.int32),
        pltpu.VMEM((G, CHUNK), jnp.int32),
        [pltpu.VMEM((CHUNK, D), jnp.float32)] * NB,
        pltpu.SemaphoreType.DMA,
        pltpu.SemaphoreType.DMA,
    ]
    if with_deg:
        out_type.append(jax.ShapeDtypeStruct((NC, R_PAD, 16), jnp.float32))
        scratch += [
            pltpu.VMEM_SHARED((R_PAD, 16), jnp.float32),
            pltpu.VMEM((CHUNK, 16), jnp.float32),
        ]

    def body(m_hbm, src_hbm, dst_hbm, *rest):
        if with_deg:
            (zc_hbm, z16_hbm, o16_hbm, acc_out, deg_out, acc_sh,
             src_v, dst_v, rows, gsem, ssem, deg_sh, ones_v) = rest
        else:
            (zc_hbm, acc_out, acc_sh, src_v, dst_v, rows, gsem, ssem) = rest
        cid = lax.axis_index("c")
        sid = lax.axis_index("s")
        base = sid * RPT

        pltpu.sync_copy(zc_hbm, rows[0])

        @pl.loop(0, RPT, step=CHUNK)
        def _(r):
            pltpu.sync_copy(rows[0], acc_sh.at[pl.ds(base + r, CHUNK)])

        if with_deg:
            pltpu.sync_copy(z16_hbm, ones_v)

            @pl.loop(0, RPT, step=CHUNK)
            def _(r):
                pltpu.sync_copy(ones_v, deg_sh.at[pl.ds(base + r, CHUNK)])

            pltpu.sync_copy(o16_hbm, ones_v)

        plsc.subcore_barrier()

        def edge_loop(c_lo, c_hi):
            @pl.loop(c_lo, c_hi, step=G)
            def _(g0):
                pltpu.sync_copy(src_hbm.at[sid, pl.ds(g0, G)], src_v)
                pltpu.sync_copy(dst_hbm.at[sid, pl.ds(g0, G)], dst_v)

                @pl.loop(0, G, step=NB)
                def _(j):
                    gathers = [
                        pltpu.async_copy(m_hbm.at[src_v.at[j + b]], rows[b],
                                         gsem)
                        for b in range(NB)]
                    scatters = []
                    for b in range(NB):
                        gathers[b].wait()
                        scatters.append(pltpu.async_copy(
                            rows[b], acc_sh.at[dst_v.at[j + b]], ssem,
                            add=True))
                        if with_deg:
                            scatters.append(pltpu.async_copy(
                                ones_v, deg_sh.at[dst_v.at[j + b]], ssem,
                                add=True))
                    for cp in scatters:
                        cp.wait()

        with jax.named_scope("edge_loop"):
            @pl.when(cid == 0)
            def _():
                edge_loop(0, CPT0)

            @pl.when(cid == 1)
            def _():
                edge_loop(CPT0, CPR)

        plsc.subcore_barrier()

        with jax.named_scope("writeback"):
            @pl.loop(0, RPT, step=CHUNK)
            def _(r):
                pltpu.sync_copy(acc_sh.at[pl.ds(base + r, CHUNK)], rows[0])
                pltpu.sync_copy(rows[0],
                                acc_out.at[cid, pl.ds(base + r, CHUNK)])

            if with_deg:
                @pl.loop(0, RPT, step=CHUNK)
                def _(r):
                    pltpu.sync_copy(deg_sh.at[pl.ds(base + r, CHUNK)], ones_v)
                    pltpu.sync_copy(ones_v,
                                    deg_out.at[cid, pl.ds(base + r, CHUNK)])

    return pl.kernel(
        body, out_type=out_type, mesh=mesh, scratch_types=scratch,
        compiler_params=pltpu.CompilerParams(use_tc_tiling_on_sc=False))


_BR = 2000


def _tc_layer_in(x, w_self, w_neigh, b):
    n = x.shape[0]

    def body(x_ref, ws_ref, wn_ref, b_ref, s_ref, m_ref):
        xb = x_ref[...]
        s_ref[...] = (jnp.dot(xb, ws_ref[...],
                              preferred_element_type=jnp.float32) + b_ref[...])
        m_ref[...] = jnp.dot(xb, wn_ref[...],
                             preferred_element_type=jnp.float32)

    return pl.pallas_call(
        body,
        grid=(n // _BR,),
        in_specs=[
            pl.BlockSpec((_BR, D), lambda i: (i, 0)),
            pl.BlockSpec((D, D), lambda i: (0, 0)),
            pl.BlockSpec((D, D), lambda i: (0, 0)),
            pl.BlockSpec((1, D), lambda i: (0, 0)),
        ],
        out_specs=[
            pl.BlockSpec((_BR, D), lambda i: (i, 0)),
            pl.BlockSpec((_BR, D), lambda i: (i, 0)),
        ],
        out_shape=[jax.ShapeDtypeStruct((n, D), jnp.float32)] * 2,
    )(x, w_self, w_neigh, b.reshape(1, D))


def _tc_layer_mid(s1, accp, degp, w_self, w_neigh, b):
    n = s1.shape[0]

    def body(s1_ref, acc_ref, deg_ref, ws_ref, wn_ref, b_ref, s_ref, m_ref):
        agg = acc_ref[0] + acc_ref[1]
        deg = deg_ref[0][:, 0:1] + deg_ref[1][:, 0:1]
        rdeg = 1.0 / jnp.maximum(deg, 1.0)
        h = jnp.maximum(s1_ref[...] + agg * rdeg, 0.0)
        s_ref[...] = (jnp.dot(h, ws_ref[...],
                              preferred_element_type=jnp.float32) + b_ref[...])
        m_ref[...] = jnp.dot(h, wn_ref[...],
                             preferred_element_type=jnp.float32)

    return pl.pallas_call(
        body,
        grid=(n // _BR,),
        in_specs=[
            pl.BlockSpec((_BR, D), lambda i: (i, 0)),
            pl.BlockSpec((NC, _BR, D), lambda i: (0, i, 0)),
            pl.BlockSpec((NC, _BR, 16), lambda i: (0, i, 0)),
            pl.BlockSpec((D, D), lambda i: (0, 0)),
            pl.BlockSpec((D, D), lambda i: (0, 0)),
            pl.BlockSpec((1, D), lambda i: (0, 0)),
        ],
        out_specs=[
            pl.BlockSpec((_BR, D), lambda i: (i, 0)),
            pl.BlockSpec((_BR, D), lambda i: (i, 0)),
        ],
        out_shape=[jax.ShapeDtypeStruct((n, D), jnp.float32)] * 2,
    )(s1, accp, degp, w_self, w_neigh, b.reshape(1, D))


def _tc_layer_out(s2, accp, degp):
    n = s2.shape[0]

    def body(s2_ref, acc_ref, deg_ref, o_ref):
        agg = acc_ref[0] + acc_ref[1]
        deg = deg_ref[0][:, 0:1] + deg_ref[1][:, 0:1]
        rdeg = 1.0 / jnp.maximum(deg, 1.0)
        o_ref[...] = s2_ref[...] + agg * rdeg

    return pl.pallas_call(
        body,
        grid=(n // _BR,),
        in_specs=[
            pl.BlockSpec((_BR, D), lambda i: (i, 0)),
            pl.BlockSpec((NC, _BR, D), lambda i: (0, i, 0)),
            pl.BlockSpec((NC, _BR, 16), lambda i: (0, i, 0)),
        ],
        out_specs=pl.BlockSpec((_BR, D), lambda i: (i, 0)),
        out_shape=jax.ShapeDtypeStruct((n, D), jnp.float32),
    )(s2, accp, degp)


def kernel(x, edge_index, W_self1, W_neigh1, b1, W_self2, W_neigh2, b2):
    src = edge_index[0].astype(jnp.int32)
    dst = edge_index[1].astype(jnp.int32)
    pad = NS * CPR * CHUNK - src.shape[0]
    pad_iota = jnp.arange(pad, dtype=jnp.int32)
    pad_src = pad_iota % N_NODES
    pad_dst = N_NODES + pad_iota % (R_PAD - N_NODES)
    src_r = jnp.concatenate([src, pad_src]).reshape(NS, CPR, CHUNK)
    dst_r = jnp.concatenate([dst, pad_dst]).reshape(NS, CPR, CHUNK)

    zc = jnp.zeros((CHUNK, D), jnp.float32)
    z16 = jnp.zeros((CHUNK, 16), jnp.float32)
    o16 = jnp.ones((CHUNK, 16), jnp.float32)

    s1, m1 = _tc_layer_in(x, W_self1, W_neigh1, b1)
    accp1, degp = _make_sc_segment_sum(True)(m1, src_r, dst_r, zc, z16, o16)
    s2, m2 = _tc_layer_mid(s1, accp1, degp, W_self2, W_neigh2, b2)
    accp2, = _make_sc_segment_sum(False)(m2, src_r, dst_r, zc)
    return _tc_layer_out(s2, accp2, degp)
